# baseline (speedup 1.0000x reference)
import functools

import jax
import jax.numpy as jnp
from jax import lax
from jax.experimental import pallas as pl
from jax.experimental.pallas import tpu as pltpu
from jax.experimental.pallas import tpu_sc as plsc

N = 10000
E = 320000
D = 128
DE = 16
H = 128
NG = H // 16

NC = 2
NS = 16
NW = NC * NS
EB = 40
E_PER_TILE = E // NW
NBLK = E_PER_TILE // EB
N_PAD = 10240


def _pq_body(x_ref, w_ref, pq_ref):
    half = pl.program_id(0) // (N // 1000)
    pq_ref[...] = jnp.dot(x_ref[...], w_ref[half],
                          preferred_element_type=jnp.float32,
                          precision=jax.lax.Precision.HIGHEST)


def _compute_pq(x, wab):
    blk = 1000
    nb = N // blk
    return pl.pallas_call(
        _pq_body,
        grid=(2 * nb,),
        in_specs=[
            pl.BlockSpec((blk, D), lambda i: (i % (N // 1000), 0)),
            pl.BlockSpec((2, D, H), lambda i: (0, 0, 0)),
        ],
        out_specs=pl.BlockSpec((blk, H), lambda i: (i, 0)),
        out_shape=jax.ShapeDtypeStruct((2 * N, H), jnp.float32),
    )(x, wab)


def _r_body(ea_ref, wc_ref, b1_ref, r_ref):
    r_ref[...] = (
        jnp.dot(ea_ref[...], wc_ref[...], preferred_element_type=jnp.float32,
                precision=jax.lax.Precision.HIGHEST)
        + b1_ref[...]
    )


def _compute_r(edge_attr, wc, b1r):
    blk = 8000
    return pl.pallas_call(
        _r_body,
        grid=(E // blk,),
        in_specs=[
            pl.BlockSpec((blk, DE), lambda i: (i, 0)),
            pl.BlockSpec((DE, H), lambda i: (0, 0)),
            pl.BlockSpec((1, H), lambda i: (0, 0)),
        ],
        out_specs=pl.BlockSpec((blk, H), lambda i: (i, 0)),
        out_shape=jax.ShapeDtypeStruct((E, H), jnp.float32),
    )(edge_attr, wc, b1r)


def _edge_body(pq_hbm, r_hbm, jdx_hbm, w2_hbm, zero_hbm,
               out_hbm, jdxi_v, pq_v, r_v, s_v, dstb_v, w2_v,
               shared, sems):
    c = lax.axis_index("c")
    s = lax.axis_index("s")
    wid = c * NS + s

    @pl.when(s == 0)
    def _():
        pltpu.sync_copy(zero_hbm, shared)

    pltpu.sync_copy(jdx_hbm.at[wid], jdxi_v)
    pltpu.sync_copy(w2_hbm, w2_v)
    plsc.subcore_barrier()

    zero16 = jnp.zeros((16,), jnp.float32)
    w2g = [w2_v[pl.ds(g * 16, 16)] for g in range(NG)]
    lanes = lax.iota(jnp.int32, 16)
    perms = [lanes ^ sh for sh in (8, 4, 2, 1)]
    shift8 = jnp.minimum(lanes + 8, 15)
    lane_lt8 = lanes < 8
    dummy_idx = jnp.full((16,), N, jnp.int32)
    row_base = wid * NBLK

    sem_pq = [sems.at[0], sems.at[1]]
    sem_r = [sems.at[2], sems.at[3]]
    sem_s = [sems.at[4], sems.at[5]]

    def _joff(b):
        return pl.multiple_of(b * 2 * EB, 8)

    def _roff(b):
        return pl.multiple_of((row_base + b) * EB, 8)

    def start(b, k):
        pltpu.async_copy(
            pq_hbm.at[jdxi_v.at[pl.ds(_joff(b), 2 * EB)]], pq_v.at[k],
            sem_pq[k])
        pltpu.async_copy(r_hbm.at[pl.ds(_roff(b), EB)], r_v.at[k], sem_r[k])

    def wait(b, k):
        pltpu.make_async_copy(
            pq_hbm.at[jdxi_v.at[pl.ds(_joff(b), 2 * EB)]], pq_v.at[k],
            sem_pq[k]).wait()
        pltpu.make_async_copy(
            r_hbm.at[pl.ds(_roff(b), EB)], r_v.at[k], sem_r[k]).wait()

    def scatter(k):
        pltpu.async_copy(s_v.at[k], shared.at[dstb_v.at[k]], sem_s[k],
                         add=True)

    def wait_scatter(k):
        pltpu.make_async_copy(
            s_v.at[k], shared.at[dstb_v.at[k]], sem_s[k]).wait()

    def compute(b, k):
        joff = _joff(b)
        dstb_v[k, pl.ds(0, 16)] = jdxi_v[pl.ds(joff, 16)]
        dstb_v[k, pl.ds(16, 16)] = jdxi_v[pl.ds(joff + 16, 16)]
        tail_raw = jdxi_v[pl.ds(joff + 24, 16)]
        tail_idx = jnp.where(lane_lt8, tail_raw[shift8], dummy_idx)
        dstb_v[k, pl.ds(32, 16)] = tail_idx

        svec = zero16
        for e in range(EB):
            acc = zero16
            for g in range(NG):
                t = (pq_v[k, e, pl.ds(g * 16, 16)]
                     + pq_v[k, EB + e, pl.ds(g * 16, 16)]
                     + r_v[k, e, pl.ds(g * 16, 16)])
                acc = acc + jnp.maximum(t, 0.0) * w2g[g]
            for prm in perms:
                acc = acc + acc[prm]
            lane = e % 16 if e < 32 else e - 32
            svec = jnp.where(lanes == lane, acc, svec)
            if e in (15, 31):
                s_v[k, pl.ds(e - 15, 16)] = svec
                svec = zero16
            elif e == EB - 1:
                svec = jnp.where(lane_lt8, svec, zero16)
                s_v[k, pl.ds(32, 16)] = svec

    start(0, 0)

    def pair(j, carry):
        i = j * 2
        start(i + 1, 1)
        wait(i, 0)

        @pl.when(j > 0)
        def _():
            wait_scatter(0)

        compute(i, 0)
        scatter(0)

        @pl.when(i + 2 < NBLK)
        def _():
            start(i + 2, 0)

        wait(i + 1, 1)

        @pl.when(j > 0)
        def _():
            wait_scatter(1)

        compute(i + 1, 1)
        scatter(1)
        return carry

    lax.fori_loop(0, NBLK // 2, pair, 0)
    wait_scatter(0)
    wait_scatter(1)

    plsc.subcore_barrier()

    @pl.when(s == 0)
    def _():
        pltpu.sync_copy(shared, out_hbm.at[c])


@functools.lru_cache(maxsize=None)
def _make_edge_kernel():
    return functools.partial(
        pl.kernel,
        out_type=jax.ShapeDtypeStruct((NC, N_PAD), jnp.float32),
        mesh=plsc.VectorSubcoreMesh(
            core_axis_name="c", subcore_axis_name="s", num_cores=NC,
            num_subcores=NS),
        scratch_types=[
            pltpu.VMEM((2 * E_PER_TILE,), jnp.int32),
            pltpu.VMEM((2, 2 * EB, H), jnp.float32),
            pltpu.VMEM((2, EB, H), jnp.float32),
            pltpu.VMEM((2, 48), jnp.float32),
            pltpu.VMEM((2, 48), jnp.int32),
            pltpu.VMEM((H,), jnp.float32),
            pltpu.VMEM_SHARED((N_PAD,), jnp.float32),
            pltpu.SemaphoreType.DMA((6,)),
        ],
    )(lambda pq, r, jdx, w2, zero, out, *scratch:
          _edge_body(pq, r, jdx, w2, zero, out, *scratch))


def _fin_body(part_ref, b2_ref, h_ref):
    h_ref[...] = jnp.sum(part_ref[...], axis=0, keepdims=True) + b2_ref[...]


def _combine(partials, b2r):
    return pl.pallas_call(
        _fin_body,
        in_specs=[
            pl.BlockSpec((NC, N_PAD), lambda: (0, 0)),
            pl.BlockSpec((1, 1), lambda: (0, 0)),
        ],
        out_specs=pl.BlockSpec((1, N_PAD), lambda: (0, 0)),
        out_shape=jax.ShapeDtypeStruct((1, N_PAD), jnp.float32),
    )(partials, b2r)


def kernel(x, edge_attr, edge_index, W1, b1, W2, b2):
    wab = jnp.stack([W1[:D], W1[D:2 * D]])
    wc = W1[2 * D:]
    pq = _compute_pq(x, wab)
    r = _compute_r(edge_attr, wc, b1.reshape(1, H))
    dst3 = edge_index[1].reshape(NW, NBLK, EB)
    src3 = edge_index[0].reshape(NW, NBLK, EB) + N
    jdx = jnp.concatenate([dst3, src3], axis=2).reshape(NW, 2 * E_PER_TILE)
    zero = jnp.zeros((N_PAD,), jnp.float32)
    partials = _make_edge_kernel()(pq, r, jdx, W2.reshape(H), zero)
    h = _combine(partials, b2.reshape(1, 1))
    return h[0, :N].reshape(N, 1)
